# Initial kernel scaffold; baseline (speedup 1.0000x reference)
import jax
import jax.numpy as jnp
from jax.experimental import pallas as pl


def kernel(x, edge_label_index):
    raise NotImplementedError("write your pallas kernel here")


# speedup vs baseline: 1.1852x; 1.1852x over previous
import functools

import jax
import jax.numpy as jnp
from jax import lax
from jax.experimental import pallas as pl
from jax.experimental.pallas import tpu as pltpu
from jax.experimental.pallas import tpu_sc as plsc

E = 320000
D = 128
NC = 2
NS = 16
NW = NC * NS
EPW = E // NW
G = 80
NCHUNK = EPW // G
L = 16

_mesh = plsc.VectorSubcoreMesh(core_axis_name="c", subcore_axis_name="s")


@functools.partial(
    pl.kernel,
    mesh=_mesh,
    out_type=jax.ShapeDtypeStruct((E,), jnp.float32),
    compiler_params=pltpu.CompilerParams(needs_layout_passes=False),
    scratch_types=[
        pltpu.VMEM((EPW,), jnp.int32),
        pltpu.VMEM((EPW,), jnp.int32),
        pltpu.VMEM((G, D), jnp.float32),
        pltpu.VMEM((G, D), jnp.float32),
        pltpu.VMEM((G,), jnp.float32),
        pltpu.SemaphoreType.DMA,
        pltpu.SemaphoreType.DMA,
    ],
)
def _ip_kernel(x_hbm, row_hbm, col_hbm, out_hbm,
               rows_v, cols_v, src_v, dst_v, out_v, sem_s, sem_d):
    wid = lax.axis_index("s") * NC + lax.axis_index("c")
    wbase = wid * EPW
    pltpu.sync_copy(row_hbm.at[pl.ds(wbase, EPW)], rows_v)
    pltpu.sync_copy(col_hbm.at[pl.ds(wbase, EPW)], cols_v)

    def chunk(g, carry):
        csl = pl.ds(g * G, G)
        cs = pltpu.async_copy(x_hbm.at[rows_v.at[csl]], src_v, sem_s)
        cd = pltpu.async_copy(x_hbm.at[cols_v.at[csl]], dst_v, sem_d)
        cs.wait()
        cd.wait()
        for e0 in range(0, G, L):
            rows = jnp.arange(e0, e0 + L, dtype=jnp.int32)

            def dstep(dd, acc):
                cidx = jnp.full((L,), dd, dtype=jnp.int32)
                sv = plsc.load_gather(src_v, [rows, cidx])
                dv = plsc.load_gather(dst_v, [rows, cidx])
                return acc + sv * dv

            acc = lax.fori_loop(0, D, dstep, jnp.zeros((L,), jnp.float32))
            out_v[pl.ds(e0, L)] = acc
        pltpu.sync_copy(out_v, out_hbm.at[pl.ds(wbase + g * G, G)])
        return carry

    lax.fori_loop(0, NCHUNK, chunk, 0)


def kernel(x, edge_label_index):
    eli = edge_label_index.astype(jnp.int32)
    out = _ip_kernel(x, eli[0], eli[1])
    return out[:, None]
